# baseline (speedup 1.0000x reference)
import functools

import jax
import jax.numpy as jnp
from jax import lax
from jax.experimental import pallas as pl
from jax.experimental.pallas import tpu as pltpu
from jax.experimental.pallas import tpu_sc as plsc


_CHUNK = 128
_NBUF = 5


@functools.lru_cache(maxsize=None)
def _make_gather(n, v, d):
    info = plsc.get_sparse_core_info()
    nc, ns = info.num_cores, info.num_subcores
    nw = nc * ns
    assert n % (nw * _CHUNK) == 0
    per_w = n // nw
    nchunks = per_w // _CHUNK
    assert nchunks % _NBUF == 0

    mesh = plsc.VectorSubcoreMesh(core_axis_name="c", subcore_axis_name="s")

    @functools.partial(
        pl.kernel,
        mesh=mesh,
        out_type=jax.ShapeDtypeStruct((n, d), jnp.float32),
        scratch_types=[
            pltpu.VMEM((per_w,), jnp.int32),
            pltpu.VMEM((_NBUF, _CHUNK, d), jnp.float32),
            pltpu.SemaphoreType.DMA((_NBUF,)),
            pltpu.SemaphoreType.DMA((_NBUF,)),
        ],
    )
    def body(x_hbm, w_hbm, out_hbm, idx_v, rows_v, gsem, wsem):
        wid = lax.axis_index("s") * nc + lax.axis_index("c")
        base = wid * per_w

        pltpu.sync_copy(x_hbm.at[pl.ds(base, per_w)], idx_v)

        def start_gather(chunk, b):
            pltpu.async_copy(
                w_hbm.at[idx_v.at[pl.ds(chunk * _CHUNK, _CHUNK)]],
                rows_v.at[b],
                gsem.at[b],
            )

        def wait_gather(b):
            pltpu.make_async_copy(
                w_hbm.at[idx_v.at[pl.ds(0, _CHUNK)]],
                rows_v.at[b],
                gsem.at[b],
            ).wait()

        def start_write(chunk, b):
            pltpu.async_copy(
                rows_v.at[b],
                out_hbm.at[pl.ds(base + chunk * _CHUNK, _CHUNK)],
                wsem.at[b],
            )

        def wait_write(b):
            pltpu.make_async_copy(
                rows_v.at[b],
                out_hbm.at[pl.ds(base, _CHUNK)],
                wsem.at[b],
            ).wait()

        for b in range(_NBUF):
            start_gather(b, b)

        def step(g, carry):
            for b in range(_NBUF):
                chunk = g * _NBUF + b
                wait_gather(b)
                start_write(chunk, b)
                nxt = chunk + _NBUF

                @pl.when(nxt < nchunks)
                def _():
                    wait_write(b)
                    start_gather(nxt, b)

            return carry

        lax.fori_loop(0, nchunks // _NBUF, step, 0)

        for b in range(_NBUF):
            wait_write(b)

    return body


def kernel(x, W):
    b, s = x.shape
    v, d = W.shape
    n = b * s
    out = _make_gather(n, v, d)(x.reshape(n), W)
    return out.reshape(b, s, d)
